# baseline (speedup 1.0000x reference)
import functools

import jax
import jax.numpy as jnp
from jax import lax
from jax.experimental import pallas as pl
from jax.experimental.pallas import tpu as pltpu
from jax.experimental.pallas import tpu_sc as plsc

_NUM_CORES = 2
_NUM_SUBCORES = 16
_NUM_WORKERS = _NUM_CORES * _NUM_SUBCORES
_CHUNK = 400


@functools.lru_cache(maxsize=None)
def _make_gather(V, D, B):
  del V
  assert B % _NUM_WORKERS == 0
  b_per_w = B // _NUM_WORKERS
  assert b_per_w % _CHUNK == 0
  n_chunks = b_per_w // _CHUNK
  mesh = plsc.VectorSubcoreMesh(core_axis_name="c", subcore_axis_name="s")

  @functools.partial(
      pl.kernel,
      mesh=mesh,
      out_type=jax.ShapeDtypeStruct((B, D), jnp.float32),
      scratch_types=[
          pltpu.VMEM((b_per_w,), jnp.int32),
          pltpu.VMEM((_CHUNK, D), jnp.float32),
          pltpu.VMEM((_CHUNK, D), jnp.float32),
          pltpu.SemaphoreType.DMA,
          pltpu.SemaphoreType.DMA,
          pltpu.SemaphoreType.DMA,
          pltpu.SemaphoreType.DMA,
      ],
  )
  def gather_kernel(table_hbm, idx_hbm, out_hbm, idx_v, rows_v0, rows_v1,
                    sg0, sg1, ss0, ss1):
    wid = lax.axis_index("s") * _NUM_CORES + lax.axis_index("c")
    base = wid * b_per_w
    rows_v = (rows_v0, rows_v1)
    sg = (sg0, sg1)
    ss = (ss0, ss1)

    pltpu.sync_copy(idx_hbm.at[pl.ds(base, b_per_w)], idx_v)

    def start_gather(i):
      b = i % 2
      return pltpu.async_copy(
          table_hbm.at[idx_v.at[pl.ds(i * _CHUNK, _CHUNK)]], rows_v[b], sg[b])

    gathers = [None] * n_chunks
    stores = [None] * n_chunks
    gathers[0] = start_gather(0)
    for i in range(n_chunks):
      b = i % 2
      if i + 1 < n_chunks:
        if i >= 1:
          stores[i - 1].wait()
        gathers[i + 1] = start_gather(i + 1)
      gathers[i].wait()
      stores[i] = pltpu.async_copy(
          rows_v[b], out_hbm.at[pl.ds(base + i * _CHUNK, _CHUNK)], ss[b])
    stores[n_chunks - 1].wait()

  return gather_kernel


def kernel(src, index):
  idx = index.astype(jnp.int32)
  return _make_gather(src.shape[0], src.shape[1], idx.shape[0])(src, idx)


# speedup vs baseline: 1.0722x; 1.0722x over previous
import functools

import jax
import jax.numpy as jnp
from jax import lax
from jax.experimental import pallas as pl
from jax.experimental.pallas import tpu as pltpu
from jax.experimental.pallas import tpu_sc as plsc

_NUM_CORES = 2
_NUM_SUBCORES = 16
_NUM_WORKERS = _NUM_CORES * _NUM_SUBCORES
_CHUNK = 80
_NBUF = 4


@functools.lru_cache(maxsize=None)
def _make_gather(V, D, B):
  assert B % _NUM_WORKERS == 0
  b_per_w = B // _NUM_WORKERS
  assert b_per_w % _CHUNK == 0
  n_chunks = b_per_w // _CHUNK
  n_bodies = n_chunks // _NBUF
  n_tail = n_chunks - n_bodies * _NBUF
  mesh = plsc.VectorSubcoreMesh(core_axis_name="c", subcore_axis_name="s")

  @functools.partial(
      pl.kernel,
      mesh=mesh,
      out_type=jax.ShapeDtypeStruct((B, D), jnp.float32),
      scratch_types=[
          pltpu.VMEM_SHARED((V, D), jnp.float32),
      ] + [pltpu.VMEM((_CHUNK,), jnp.int32) for _ in range(_NBUF)]
        + [pltpu.VMEM((_CHUNK, D), jnp.float32) for _ in range(_NBUF)]
        + [pltpu.SemaphoreType.DMA for _ in range(2 * _NBUF)],
  )
  def gather_kernel(table_hbm, idx_hbm, out_hbm, table_sh, *bufs):
    idx_v = bufs[:_NBUF]
    rows_v = bufs[_NBUF:2 * _NBUF]
    sg = bufs[2 * _NBUF:3 * _NBUF]
    ss = bufs[3 * _NBUF:4 * _NBUF]
    sid = lax.axis_index("s")
    wid = sid * _NUM_CORES + lax.axis_index("c")
    base = wid * b_per_w

    @pl.when(sid == 0)
    def _():
      pltpu.sync_copy(table_hbm, table_sh)

    plsc.subcore_barrier()

    def do_group(first_chunk):
      gathers = []
      for b in range(_NBUF):
        off = base + (first_chunk + b) * _CHUNK
        pltpu.sync_copy(idx_hbm.at[pl.ds(off, _CHUNK)], idx_v[b])
        gathers.append(
            pltpu.async_copy(table_sh.at[idx_v[b]], rows_v[b], sg[b]))
      stores = []
      for b in range(_NBUF):
        off = base + (first_chunk + b) * _CHUNK
        gathers[b].wait()
        stores.append(
            pltpu.async_copy(rows_v[b], out_hbm.at[pl.ds(off, _CHUNK)],
                             ss[b]))
      for b in range(_NBUF):
        stores[b].wait()

    def body(j, carry):
      do_group(j * _NBUF)
      return carry

    lax.fori_loop(0, n_bodies, body, None)

    for t in range(n_tail):
      c = n_bodies * _NBUF + t
      off = base + c * _CHUNK
      pltpu.sync_copy(idx_hbm.at[pl.ds(off, _CHUNK)], idx_v[0])
      pltpu.async_copy(table_sh.at[idx_v[0]], rows_v[0], sg[0]).wait()
      pltpu.sync_copy(rows_v[0], out_hbm.at[pl.ds(off, _CHUNK)])

  return gather_kernel


def kernel(src, index):
  idx = index.astype(jnp.int32)
  return _make_gather(src.shape[0], src.shape[1], idx.shape[0])(src, idx)


# speedup vs baseline: 1.3246x; 1.2353x over previous
import functools

import jax
import jax.numpy as jnp
from jax import lax
from jax.experimental import pallas as pl
from jax.experimental.pallas import tpu as pltpu
from jax.experimental.pallas import tpu_sc as plsc

_NUM_CORES = 2
_NUM_SUBCORES = 16
_NUM_WORKERS = _NUM_CORES * _NUM_SUBCORES
_CHUNK = 80
_NBUF = 4


@functools.lru_cache(maxsize=None)
def _make_gather(V, D, B):
  assert B % _NUM_WORKERS == 0
  b_per_w = B // _NUM_WORKERS
  assert b_per_w % _CHUNK == 0
  n_chunks = b_per_w // _CHUNK
  n_bodies = n_chunks // _NBUF
  n_tail = n_chunks - n_bodies * _NBUF
  mesh = plsc.VectorSubcoreMesh(core_axis_name="c", subcore_axis_name="s")

  @functools.partial(
      pl.kernel,
      mesh=mesh,
      out_type=jax.ShapeDtypeStruct((B, D), jnp.float32),
      scratch_types=[
          pltpu.VMEM_SHARED((V, D), jnp.float32),
          pltpu.VMEM((b_per_w,), jnp.int32),
      ] + [pltpu.VMEM((_CHUNK, D), jnp.float32) for _ in range(_NBUF)]
        + [pltpu.SemaphoreType.DMA for _ in range(2 * _NBUF)],
  )
  def gather_kernel(table_hbm, idx_hbm, out_hbm, table_sh, idx_v, *bufs):
    rows_v = bufs[:_NBUF]
    sg = bufs[_NBUF:2 * _NBUF]
    ss = bufs[2 * _NBUF:3 * _NBUF]
    sid = lax.axis_index("s")
    wid = sid * _NUM_CORES + lax.axis_index("c")
    base = wid * b_per_w

    pltpu.sync_copy(idx_hbm.at[pl.ds(base, b_per_w)], idx_v)

    @pl.when(sid == 0)
    def _():
      pltpu.sync_copy(table_hbm, table_sh)

    plsc.subcore_barrier()

    def do_group(first_chunk):
      gathers = []
      for b in range(_NBUF):
        gathers.append(
            pltpu.async_copy(
                table_sh.at[idx_v.at[pl.ds((first_chunk + b) * _CHUNK,
                                           _CHUNK)]], rows_v[b], sg[b]))
      stores = []
      for b in range(_NBUF):
        off = base + (first_chunk + b) * _CHUNK
        gathers[b].wait()
        stores.append(
            pltpu.async_copy(rows_v[b], out_hbm.at[pl.ds(off, _CHUNK)],
                             ss[b]))
      for b in range(_NBUF):
        stores[b].wait()

    def body(j, carry):
      do_group(j * _NBUF)
      return carry

    lax.fori_loop(0, n_bodies, body, None)

    for t in range(n_tail):
      c = n_bodies * _NBUF + t
      off = base + c * _CHUNK
      pltpu.async_copy(table_sh.at[idx_v.at[pl.ds(c * _CHUNK, _CHUNK)]],
                       rows_v[0], sg[0]).wait()
      pltpu.sync_copy(rows_v[0], out_hbm.at[pl.ds(off, _CHUNK)])

  return gather_kernel


def kernel(src, index):
  idx = index.astype(jnp.int32)
  return _make_gather(src.shape[0], src.shape[1], idx.shape[0])(src, idx)


# speedup vs baseline: 1.5798x; 1.1927x over previous
import functools

import jax
import jax.numpy as jnp
from jax import lax
from jax.experimental import pallas as pl
from jax.experimental.pallas import tpu as pltpu
from jax.experimental.pallas import tpu_sc as plsc

_NUM_CORES = 2
_NUM_SUBCORES = 16
_NUM_WORKERS = _NUM_CORES * _NUM_SUBCORES
_CHUNK = 80
_NBUF = 4


@functools.lru_cache(maxsize=None)
def _make_gather(V, D, B):
  assert B % _NUM_WORKERS == 0
  b_per_w = B // _NUM_WORKERS
  assert b_per_w % _CHUNK == 0
  n_chunks = b_per_w // _CHUNK
  n_bodies = n_chunks // _NBUF
  n_tail = n_chunks - n_bodies * _NBUF
  mesh = plsc.VectorSubcoreMesh(core_axis_name="c", subcore_axis_name="s")

  @functools.partial(
      pl.kernel,
      mesh=mesh,
      out_type=jax.ShapeDtypeStruct((B, D), jnp.float32),
      scratch_types=[
          pltpu.VMEM_SHARED((V, D), jnp.float32),
          pltpu.VMEM((b_per_w,), jnp.int32),
      ] + [pltpu.VMEM((_CHUNK, D), jnp.float32) for _ in range(_NBUF)]
        + [pltpu.SemaphoreType.DMA for _ in range(2 * _NBUF)],
  )
  def gather_kernel(table_hbm, idx_hbm, out_hbm, table_sh, idx_v, *bufs):
    rows_v = bufs[:_NBUF]
    sg = bufs[_NBUF:2 * _NBUF]
    ss = bufs[2 * _NBUF:3 * _NBUF]
    sid = lax.axis_index("s")
    wid = sid * _NUM_CORES + lax.axis_index("c")
    base = wid * b_per_w

    even = (V // _NUM_SUBCORES) // 8 * 8
    last = V - (_NUM_SUBCORES - 1) * even
    pltpu.sync_copy(idx_hbm.at[pl.ds(base, b_per_w)], idx_v)

    @pl.when(sid < _NUM_SUBCORES - 1)
    def _():
      pltpu.sync_copy(table_hbm.at[pl.ds(sid * even, even)],
                      table_sh.at[pl.ds(sid * even, even)])

    @pl.when(sid == _NUM_SUBCORES - 1)
    def _():
      pltpu.sync_copy(table_hbm.at[pl.ds((_NUM_SUBCORES - 1) * even, last)],
                      table_sh.at[pl.ds((_NUM_SUBCORES - 1) * even, last)])

    plsc.subcore_barrier()

    def wait_store(b, chunk):
      pltpu.make_async_copy(
          rows_v[b], out_hbm.at[pl.ds(base + chunk * _CHUNK, _CHUNK)],
          ss[b]).wait()

    def start_store(b, chunk):
      return pltpu.async_copy(
          rows_v[b], out_hbm.at[pl.ds(base + chunk * _CHUNK, _CHUNK)], ss[b])

    def start_gather(b, chunk):
      return pltpu.async_copy(
          table_sh.at[idx_v.at[pl.ds(chunk * _CHUNK, _CHUNK)]], rows_v[b],
          sg[b])

    def body(j, carry):
      gathers = []
      for b in range(_NBUF):
        @pl.when(j > 0)
        def _(b=b):
          wait_store(b, (j - 1) * _NBUF + b)

        gathers.append(start_gather(b, j * _NBUF + b))
      for b in range(_NBUF):
        gathers[b].wait()
        start_store(b, j * _NBUF + b)
      return carry

    lax.fori_loop(0, n_bodies, body, None)
    for b in range(_NBUF):
      wait_store(b, (n_bodies - 1) * _NBUF + b)

    for t in range(n_tail):
      c = n_bodies * _NBUF + t
      off = base + c * _CHUNK
      start_gather(0, c).wait()
      pltpu.sync_copy(rows_v[0], out_hbm.at[pl.ds(off, _CHUNK)])

  return gather_kernel


def kernel(src, index):
  idx = index.astype(jnp.int32)
  return _make_gather(src.shape[0], src.shape[1], idx.shape[0])(src, idx)


# speedup vs baseline: 1.5890x; 1.0058x over previous
import functools

import jax
import jax.numpy as jnp
from jax import lax
from jax.experimental import pallas as pl
from jax.experimental.pallas import tpu as pltpu
from jax.experimental.pallas import tpu_sc as plsc

_NUM_CORES = 2
_NUM_SUBCORES = 16
_NUM_WORKERS = _NUM_CORES * _NUM_SUBCORES
_CHUNK = 80
_NBUF = 4


@functools.lru_cache(maxsize=None)
def _make_gather(V, D, B):
  assert B % _NUM_WORKERS == 0
  b_per_w = B // _NUM_WORKERS
  assert b_per_w % _CHUNK == 0
  n_chunks = b_per_w // _CHUNK
  n_bodies = n_chunks // _NBUF
  n_tail = n_chunks - n_bodies * _NBUF
  mesh = plsc.VectorSubcoreMesh(core_axis_name="c", subcore_axis_name="s")

  @functools.partial(
      pl.kernel,
      mesh=mesh,
      out_type=jax.ShapeDtypeStruct((B, D), jnp.float32),
      scratch_types=[
          pltpu.VMEM_SHARED((V, D), jnp.float32),
          pltpu.VMEM((b_per_w,), jnp.int32),
      ] + [pltpu.VMEM((_CHUNK, D), jnp.float32) for _ in range(_NBUF)]
        + [pltpu.SemaphoreType.DMA for _ in range(2 * _NBUF)],
  )
  def gather_kernel(table_hbm, idx_hbm, out_hbm, table_sh, idx_v, *bufs):
    rows_v = bufs[:_NBUF]
    sg = bufs[_NBUF:2 * _NBUF]
    ss = bufs[2 * _NBUF:3 * _NBUF]
    sid = lax.axis_index("s")
    wid = sid * _NUM_CORES + lax.axis_index("c")
    base = wid * b_per_w

    even = (V // _NUM_SUBCORES) // 8 * 8
    last = V - (_NUM_SUBCORES - 1) * even
    pltpu.sync_copy(idx_hbm.at[pl.ds(base, b_per_w)], idx_v)

    first_gathers = [
        pltpu.async_copy(
            table_hbm.at[idx_v.at[pl.ds(b * _CHUNK, _CHUNK)]], rows_v[b],
            sg[b]) for b in range(_NBUF)
    ]

    @pl.when(sid < _NUM_SUBCORES - 1)
    def _():
      pltpu.sync_copy(table_hbm.at[pl.ds(sid * even, even)],
                      table_sh.at[pl.ds(sid * even, even)])

    @pl.when(sid == _NUM_SUBCORES - 1)
    def _():
      pltpu.sync_copy(table_hbm.at[pl.ds((_NUM_SUBCORES - 1) * even, last)],
                      table_sh.at[pl.ds((_NUM_SUBCORES - 1) * even, last)])

    def wait_store(b, chunk):
      pltpu.make_async_copy(
          rows_v[b], out_hbm.at[pl.ds(base + chunk * _CHUNK, _CHUNK)],
          ss[b]).wait()

    def start_store(b, chunk):
      return pltpu.async_copy(
          rows_v[b], out_hbm.at[pl.ds(base + chunk * _CHUNK, _CHUNK)], ss[b])

    def start_gather(b, chunk):
      return pltpu.async_copy(
          table_sh.at[idx_v.at[pl.ds(chunk * _CHUNK, _CHUNK)]], rows_v[b],
          sg[b])

    for b in range(_NBUF):
      first_gathers[b].wait()
      start_store(b, b)
    plsc.subcore_barrier()

    def body(j, carry):
      gathers = []
      for b in range(_NBUF):
        wait_store(b, (j - 1) * _NBUF + b)
        gathers.append(start_gather(b, j * _NBUF + b))
      for b in range(_NBUF):
        gathers[b].wait()
        start_store(b, j * _NBUF + b)
      return carry

    lax.fori_loop(1, n_bodies, body, None)
    for b in range(_NBUF):
      wait_store(b, (n_bodies - 1) * _NBUF + b)

    for t in range(n_tail):
      c = n_bodies * _NBUF + t
      off = base + c * _CHUNK
      start_gather(0, c).wait()
      pltpu.sync_copy(rows_v[0], out_hbm.at[pl.ds(off, _CHUNK)])

  return gather_kernel


def kernel(src, index):
  idx = index.astype(jnp.int32)
  return _make_gather(src.shape[0], src.shape[1], idx.shape[0])(src, idx)


# speedup vs baseline: 1.6032x; 1.0090x over previous
import functools

import jax
import jax.numpy as jnp
from jax import lax
from jax.experimental import pallas as pl
from jax.experimental.pallas import tpu as pltpu
from jax.experimental.pallas import tpu_sc as plsc

_NUM_CORES = 2
_NUM_SUBCORES = 16
_NUM_WORKERS = _NUM_CORES * _NUM_SUBCORES
_CHUNK = 40
_NBUF = 8


@functools.lru_cache(maxsize=None)
def _make_gather(V, D, B):
  assert B % _NUM_WORKERS == 0
  b_per_w = B // _NUM_WORKERS
  assert b_per_w % _CHUNK == 0
  n_chunks = b_per_w // _CHUNK
  n_bodies = n_chunks // _NBUF
  n_tail = n_chunks - n_bodies * _NBUF
  mesh = plsc.VectorSubcoreMesh(core_axis_name="c", subcore_axis_name="s")

  @functools.partial(
      pl.kernel,
      mesh=mesh,
      out_type=jax.ShapeDtypeStruct((B, D), jnp.float32),
      scratch_types=[
          pltpu.VMEM_SHARED((V, D), jnp.float32),
          pltpu.VMEM((b_per_w,), jnp.int32),
      ] + [pltpu.VMEM((_CHUNK, D), jnp.float32) for _ in range(_NBUF)]
        + [pltpu.SemaphoreType.DMA for _ in range(2 * _NBUF)],
  )
  def gather_kernel(table_hbm, idx_hbm, out_hbm, table_sh, idx_v, *bufs):
    rows_v = bufs[:_NBUF]
    sg = bufs[_NBUF:2 * _NBUF]
    ss = bufs[2 * _NBUF:3 * _NBUF]
    sid = lax.axis_index("s")
    wid = sid * _NUM_CORES + lax.axis_index("c")
    base = wid * b_per_w

    even = (V // _NUM_SUBCORES) // 8 * 8
    last = V - (_NUM_SUBCORES - 1) * even
    pltpu.sync_copy(idx_hbm.at[pl.ds(base, b_per_w)], idx_v)

    first_gathers = [
        pltpu.async_copy(
            table_hbm.at[idx_v.at[pl.ds(b * _CHUNK, _CHUNK)]], rows_v[b],
            sg[b]) for b in range(_NBUF)
    ]

    @pl.when(sid < _NUM_SUBCORES - 1)
    def _():
      pltpu.sync_copy(table_hbm.at[pl.ds(sid * even, even)],
                      table_sh.at[pl.ds(sid * even, even)])

    @pl.when(sid == _NUM_SUBCORES - 1)
    def _():
      pltpu.sync_copy(table_hbm.at[pl.ds((_NUM_SUBCORES - 1) * even, last)],
                      table_sh.at[pl.ds((_NUM_SUBCORES - 1) * even, last)])

    def wait_store(b, chunk):
      pltpu.make_async_copy(
          rows_v[b], out_hbm.at[pl.ds(base + chunk * _CHUNK, _CHUNK)],
          ss[b]).wait()

    def start_store(b, chunk):
      return pltpu.async_copy(
          rows_v[b], out_hbm.at[pl.ds(base + chunk * _CHUNK, _CHUNK)], ss[b])

    def start_gather(b, chunk):
      return pltpu.async_copy(
          table_sh.at[idx_v.at[pl.ds(chunk * _CHUNK, _CHUNK)]], rows_v[b],
          sg[b])

    for b in range(_NBUF):
      first_gathers[b].wait()
      start_store(b, b)
    plsc.subcore_barrier()

    def body(j, carry):
      gathers = []
      for b in range(_NBUF):
        wait_store(b, (j - 1) * _NBUF + b)
        gathers.append(start_gather(b, j * _NBUF + b))
      for b in range(_NBUF):
        gathers[b].wait()
        start_store(b, j * _NBUF + b)
      return carry

    lax.fori_loop(1, n_bodies, body, None)
    for b in range(_NBUF):
      wait_store(b, (n_bodies - 1) * _NBUF + b)

    for t in range(n_tail):
      c = n_bodies * _NBUF + t
      off = base + c * _CHUNK
      start_gather(0, c).wait()
      pltpu.sync_copy(rows_v[0], out_hbm.at[pl.ds(off, _CHUNK)])

  return gather_kernel


def kernel(src, index):
  idx = index.astype(jnp.int32)
  return _make_gather(src.shape[0], src.shape[1], idx.shape[0])(src, idx)
